# baseline (speedup 1.0000x reference)
import functools

import jax
import jax.numpy as jnp
from jax import lax
from jax.experimental import pallas as pl
from jax.experimental.pallas import tpu as pltpu
from jax.experimental.pallas import tpu_sc as plsc

N = 10000
E = 320000
C = 128
NC = 2
NS = 16
NPAD = 10240
ROWS_PER_TILE = NPAD // NS
EDGES_PER_TILE = E // NS
CH = 80
CHN = EDGES_PER_TILE // CH
CHB = 10
NSTAGE = CHN // CHB

_MESH = dict(core_axis_name="c", subcore_axis_name="s", num_cores=NC,
             num_subcores=NS)


@functools.partial(
    pl.kernel,
    out_type=jax.ShapeDtypeStruct((NC * NS * NPAD,), jnp.float32),
    mesh=plsc.VectorSubcoreMesh(**_MESH),
    scratch_types=[
        pltpu.VMEM((EDGES_PER_TILE,), jnp.int32),
        pltpu.VMEM((EDGES_PER_TILE,), jnp.float32),
        pltpu.VMEM((NPAD,), jnp.float32),
    ],
    compiler_params=pltpu.CompilerParams(needs_layout_passes=False),
)
def _deg_kernel(eiflat_hbm, ew_hbm, degp_hbm, idx_v, w_v, acc_v):
    c = lax.axis_index("c")
    s = lax.axis_index("s")
    base = s * EDGES_PER_TILE

    def zero(i, _):
        acc_v[pl.ds(i * 16, 16)] = jnp.zeros((16,), jnp.float32)
        return 0
    lax.fori_loop(0, NPAD // 16, zero, 0)

    pltpu.sync_copy(eiflat_hbm.at[pl.ds(c * E + base, EDGES_PER_TILE)], idx_v)
    pltpu.sync_copy(ew_hbm.at[pl.ds(base, EDGES_PER_TILE)], w_v)

    def body(i, _):
        idx = idx_v[pl.ds(i * 16, 16)]
        w = w_v[pl.ds(i * 16, 16)]
        plsc.addupdate_scatter(acc_v, [idx], w)
        return 0
    lax.fori_loop(0, EDGES_PER_TILE // 16, body, 0)

    pltpu.sync_copy(acc_v,
                    degp_hbm.at[pl.ds((c * NS + s) * NPAD, NPAD)])


def _prep_body(degp_ref, x_ref, ycat_ref, invin_ref):
    deg = jnp.sum(degp_ref[...], axis=1)
    dgo = deg[0, :N]
    dgi = deg[1, :N]
    invo = jnp.where(dgo != 0.0, 1.0 / dgo, 0.0)
    invi = jnp.where(dgi != 0.0, 1.0 / dgi, 0.0)
    xv = x_ref[...]
    ycat_ref[0:N, :] = xv * invo[:, None]
    ycat_ref[N:2 * N, :] = xv
    invin_ref[...] = jnp.broadcast_to(invi[:, None], (N, C))


def _prep_call(degp, x):
    return pl.pallas_call(
        _prep_body,
        out_shape=[jax.ShapeDtypeStruct((2 * N, C), jnp.float32),
                   jax.ShapeDtypeStruct((N, C), jnp.float32)],
    )(degp, x)


@functools.partial(
    pl.kernel,
    out_type=jax.ShapeDtypeStruct((NC, NPAD, C), jnp.float32),
    mesh=plsc.VectorSubcoreMesh(**_MESH),
    scratch_types=[
        pltpu.VMEM_SHARED((NPAD, C), jnp.float32),
        pltpu.VMEM((CHB, 2, CH), jnp.int32),
        pltpu.VMEM((CH, C), jnp.float32),
        pltpu.VMEM((CH, C), jnp.float32),
        pltpu.VMEM((16, C), jnp.float32),
        pltpu.SemaphoreType.DMA,
        pltpu.SemaphoreType.DMA,
    ],
    compiler_params=pltpu.CompilerParams(use_tc_tiling_on_sc=False),
)
def _scatter_kernel(ycat_hbm, eidx_hbm, out_hbm,
                    acc, idxv, rbufa, rbufb, zb, sema, semb):
    c = lax.axis_index("c")
    s = lax.axis_index("s")

    def z16(i, _):
        r = i // (C // 16)
        f = (i % (C // 16)) * 16
        zb[r, pl.ds(f, 16)] = jnp.zeros((16,), jnp.float32)
        return 0
    lax.fori_loop(0, 16 * (C // 16), z16, 0)

    def zc(k, _):
        pltpu.sync_copy(zb, acc.at[pl.ds(s * ROWS_PER_TILE + k * 16, 16)])
        return 0
    lax.fori_loop(0, ROWS_PER_TILE // 16, zc, 0)

    plsc.subcore_barrier()

    bufs = (rbufa, rbufb)
    sems = (sema, semb)

    def stage(g, _):
        pltpu.sync_copy(eidx_hbm.at[c, s, pl.ds(g * CHB, CHB)], idxv)

        pend = pltpu.async_copy(ycat_hbm.at[idxv.at[0, 0]], bufs[0], sems[0])
        for k in range(CHB):
            nxt = None
            if k + 1 < CHB:
                nxt = pltpu.async_copy(ycat_hbm.at[idxv.at[k + 1, 0]],
                                       bufs[(k + 1) % 2], sems[(k + 1) % 2])
            pend.wait()
            pltpu.sync_copy(bufs[k % 2], acc.at[idxv.at[k, 1]], add=True)
            pend = nxt
        return 0
    lax.fori_loop(0, NSTAGE, stage, 0)
    plsc.subcore_barrier()

    pltpu.sync_copy(acc.at[pl.ds(s * ROWS_PER_TILE, ROWS_PER_TILE)],
                    out_hbm.at[c, pl.ds(s * ROWS_PER_TILE, ROWS_PER_TILE)])


_R = 2000


def _mm_body(x_ref, to_ref, s_ref, invin_ref, w_ref, b_ref, lw_ref, lb_ref,
             o_ref):
    xb = x_ref[...]
    tob = to_ref[0]
    tib = s_ref[0] * invin_ref[...]
    w = w_ref[...]
    dot = functools.partial(jnp.dot, preferred_element_type=jnp.float32)
    az = (dot(xb, w[0] + w[1]) + dot(tob, w[2]) + dot(tib, w[3])
          + b_ref[0][None, :])
    ah = (dot(xb, w[4] + w[5]) + dot(tob, w[6]) + dot(tib, w[7])
          + b_ref[1][None, :])
    z = jax.nn.sigmoid(az)
    h = jnp.maximum((1.0 - z) * jnp.tanh(ah), 0.0)
    o_ref[...] = lax.dot_general(h, lw_ref[...], (((1,), (1,)), ((), ())),
                                 preferred_element_type=jnp.float32) \
        + lb_ref[0][None, :]


def _mm_call(x, T, invin, wstack, bstack, lin_W, lin_b):
    grid = N // _R
    return pl.pallas_call(
        _mm_body,
        grid=(grid,),
        in_specs=[
            pl.BlockSpec((_R, C), lambda i: (i, 0)),
            pl.BlockSpec((1, _R, C), lambda i: (0, i, 0)),
            pl.BlockSpec((1, _R, C), lambda i: (1, i, 0)),
            pl.BlockSpec((_R, C), lambda i: (i, 0)),
            pl.BlockSpec((8, C, C), lambda i: (0, 0, 0)),
            pl.BlockSpec((2, C), lambda i: (0, 0)),
            pl.BlockSpec((C, C), lambda i: (0, 0)),
            pl.BlockSpec((1, C), lambda i: (0, 0)),
        ],
        out_specs=pl.BlockSpec((_R, C), lambda i: (i, 0)),
        out_shape=jax.ShapeDtypeStruct((N, C), jnp.float32),
    )(x, T, T, invin, wstack, bstack, lin_W, lin_b)


def kernel(x, edge_index, edge_weight, W_z, b_z, W_r, b_r, W_h, b_h,
           lin_W, lin_b):
    del W_r, b_r
    row_r = edge_index[0].reshape(NS, CHN, CH)
    col_r = edge_index[1].reshape(NS, CHN, CH)
    eidx = jnp.stack([jnp.stack([row_r, col_r], axis=2),
                      jnp.stack([row_r + N, col_r], axis=2)])

    degp = _deg_kernel(edge_index.reshape(-1),
                       edge_weight).reshape(NC, NS, NPAD)
    ycat, invin = _prep_call(degp, x)
    T = _scatter_kernel(ycat, eidx)

    ic = slice(0, C)
    wstack = jnp.stack([
        W_z[0, 0, ic], W_z[1, 0, ic], W_z[0, 1, ic], W_z[1, 1, ic],
        W_h[0, 0, ic], W_h[1, 0, ic], W_h[0, 1, ic], W_h[1, 1, ic],
    ])
    bstack = jnp.stack([b_z, b_h])
    return _mm_call(x, T, invin, wstack, bstack, lin_W,
                    lin_b.reshape(1, C))


# speedup vs baseline: 1.1878x; 1.1878x over previous
import functools

import jax
import jax.numpy as jnp
from jax import lax
from jax.experimental import pallas as pl
from jax.experimental.pallas import tpu as pltpu
from jax.experimental.pallas import tpu_sc as plsc

N = 10000
E = 320000
C = 128
NC = 2
NS = 16
NPAD = 10240
ROWS_PER_TILE = NPAD // NS
EDGES_PER_TILE = E // NS
CH = 80
CHN = EDGES_PER_TILE // CH
CHB = 10
NSTAGE = CHN // CHB

_MESH = dict(core_axis_name="c", subcore_axis_name="s", num_cores=NC,
             num_subcores=NS)


@functools.partial(
    pl.kernel,
    out_type=jax.ShapeDtypeStruct((NC * NS * NPAD,), jnp.float32),
    mesh=plsc.VectorSubcoreMesh(**_MESH),
    scratch_types=[
        pltpu.VMEM((EDGES_PER_TILE,), jnp.int32),
        pltpu.VMEM((EDGES_PER_TILE,), jnp.float32),
        pltpu.VMEM((NPAD,), jnp.float32),
    ],
    compiler_params=pltpu.CompilerParams(needs_layout_passes=False),
)
def _deg_kernel(eiflat_hbm, ew_hbm, degp_hbm, idx_v, w_v, acc_v):
    c = lax.axis_index("c")
    s = lax.axis_index("s")
    base = s * EDGES_PER_TILE

    def zero(i, _):
        acc_v[pl.ds(i * 16, 16)] = jnp.zeros((16,), jnp.float32)
        return 0
    lax.fori_loop(0, NPAD // 16, zero, 0)

    pltpu.sync_copy(eiflat_hbm.at[pl.ds(c * E + base, EDGES_PER_TILE)], idx_v)
    pltpu.sync_copy(ew_hbm.at[pl.ds(base, EDGES_PER_TILE)], w_v)

    def body(i, _):
        idx = idx_v[pl.ds(i * 16, 16)]
        w = w_v[pl.ds(i * 16, 16)]
        plsc.addupdate_scatter(acc_v, [idx], w)
        return 0
    lax.fori_loop(0, EDGES_PER_TILE // 16, body, 0)

    pltpu.sync_copy(acc_v,
                    degp_hbm.at[pl.ds((c * NS + s) * NPAD, NPAD)])


def _prep_body(degp_ref, x_ref, ycat_ref, invin_ref):
    deg = jnp.sum(degp_ref[...], axis=1)
    dgo = deg[0, :N]
    dgi = deg[1, :N]
    invo = jnp.where(dgo != 0.0, 1.0 / dgo, 0.0)
    invi = jnp.where(dgi != 0.0, 1.0 / dgi, 0.0)
    xv = x_ref[...]
    ycat_ref[0:N, :] = xv * invo[:, None]
    ycat_ref[N:2 * N, :] = xv
    invin_ref[...] = jnp.broadcast_to(invi[:, None], (N, C))


def _prep_call(degp, x):
    return pl.pallas_call(
        _prep_body,
        out_shape=[jax.ShapeDtypeStruct((2 * N, C), jnp.float32),
                   jax.ShapeDtypeStruct((N, C), jnp.float32)],
    )(degp, x)


@functools.partial(
    pl.kernel,
    out_type=jax.ShapeDtypeStruct((NC, NPAD, C), jnp.float32),
    mesh=plsc.VectorSubcoreMesh(**_MESH),
    scratch_types=[
        pltpu.VMEM_SHARED((NPAD, C), jnp.float32),
        pltpu.VMEM((CHB, CH), jnp.int32),
        pltpu.VMEM((CHB, CH), jnp.int32),
        pltpu.VMEM((CH, C), jnp.float32),
        pltpu.VMEM((CH, C), jnp.float32),
        pltpu.VMEM((CH, C), jnp.float32),
        pltpu.VMEM((16, C), jnp.float32),
        pltpu.SemaphoreType.DMA,
        pltpu.SemaphoreType.DMA,
        pltpu.SemaphoreType.DMA,
        pltpu.SemaphoreType.DMA,
        pltpu.SemaphoreType.DMA,
        pltpu.SemaphoreType.DMA,
    ],
    compiler_params=pltpu.CompilerParams(use_tc_tiling_on_sc=False),
)
def _scatter_kernel(ycat_hbm, ridx_hbm, col_hbm, out_hbm,
                    acc, rowi, coli, rbufa, rbufb, rbufc, zb,
                    gsa, gsb, gsc, ssa, ssb, ssc):
    c = lax.axis_index("c")
    s = lax.axis_index("s")

    def z16(i, _):
        r = i // (C // 16)
        f = (i % (C // 16)) * 16
        zb[r, pl.ds(f, 16)] = jnp.zeros((16,), jnp.float32)
        return 0
    lax.fori_loop(0, 16 * (C // 16), z16, 0)

    def zc(k, _):
        pltpu.sync_copy(zb, acc.at[pl.ds(s * ROWS_PER_TILE + k * 16, 16)])
        return 0
    lax.fori_loop(0, ROWS_PER_TILE // 16, zc, 0)

    plsc.subcore_barrier()

    bufs = (rbufa, rbufb, rbufc)
    gsems = (gsa, gsb, gsc)
    ssems = (ssa, ssb, ssc)

    def stage(g, _):
        pltpu.sync_copy(ridx_hbm.at[c, s, pl.ds(g * CHB, CHB)], rowi)
        pltpu.sync_copy(col_hbm.at[s, pl.ds(g * CHB, CHB)], coli)

        gd, sd = {}, {}

        def scat(j):
            gd[j].wait()
            sd[j] = pltpu.async_copy(bufs[j % 3], acc.at[coli.at[j]],
                                     ssems[j % 3], add=True)

        for k in range(CHB):
            if k - 3 >= 0:
                sd[k - 3].wait()
            gd[k] = pltpu.async_copy(ycat_hbm.at[rowi.at[k]],
                                     bufs[k % 3], gsems[k % 3])
            if k - 2 >= 0:
                scat(k - 2)
        scat(CHB - 2)
        scat(CHB - 1)
        for j in range(CHB - 3, CHB):
            sd[j].wait()
        return 0
    lax.fori_loop(0, NSTAGE, stage, 0)
    plsc.subcore_barrier()

    pltpu.sync_copy(acc.at[pl.ds(s * ROWS_PER_TILE, ROWS_PER_TILE)],
                    out_hbm.at[c, pl.ds(s * ROWS_PER_TILE, ROWS_PER_TILE)])


_R = 2000


def _mm_body(x_ref, to_ref, s_ref, invin_ref, w_ref, b_ref, lw_ref, lb_ref,
             o_ref):
    xb = x_ref[...]
    tob = to_ref[0]
    tib = s_ref[0] * invin_ref[...]
    w = w_ref[...]
    dot = functools.partial(jnp.dot, preferred_element_type=jnp.float32)
    az = (dot(xb, w[0] + w[1]) + dot(tob, w[2]) + dot(tib, w[3])
          + b_ref[0][None, :])
    ah = (dot(xb, w[4] + w[5]) + dot(tob, w[6]) + dot(tib, w[7])
          + b_ref[1][None, :])
    z = jax.nn.sigmoid(az)
    h = jnp.maximum((1.0 - z) * jnp.tanh(ah), 0.0)
    o_ref[...] = lax.dot_general(h, lw_ref[...], (((1,), (1,)), ((), ())),
                                 preferred_element_type=jnp.float32) \
        + lb_ref[0][None, :]


def _mm_call(x, T, invin, wstack, bstack, lin_W, lin_b):
    grid = N // _R
    return pl.pallas_call(
        _mm_body,
        grid=(grid,),
        in_specs=[
            pl.BlockSpec((_R, C), lambda i: (i, 0)),
            pl.BlockSpec((1, _R, C), lambda i: (0, i, 0)),
            pl.BlockSpec((1, _R, C), lambda i: (1, i, 0)),
            pl.BlockSpec((_R, C), lambda i: (i, 0)),
            pl.BlockSpec((8, C, C), lambda i: (0, 0, 0)),
            pl.BlockSpec((2, C), lambda i: (0, 0)),
            pl.BlockSpec((C, C), lambda i: (0, 0)),
            pl.BlockSpec((1, C), lambda i: (0, 0)),
        ],
        out_specs=pl.BlockSpec((_R, C), lambda i: (i, 0)),
        out_shape=jax.ShapeDtypeStruct((N, C), jnp.float32),
    )(x, T, T, invin, wstack, bstack, lin_W, lin_b)


def kernel(x, edge_index, edge_weight, W_z, b_z, W_r, b_r, W_h, b_h,
           lin_W, lin_b):
    del W_r, b_r
    row_r = edge_index[0].reshape(NS, CHN, CH)
    col_r = edge_index[1].reshape(NS, CHN, CH)
    ridx = jnp.stack([row_r, row_r + N])

    degp = _deg_kernel(edge_index.reshape(-1),
                       edge_weight).reshape(NC, NS, NPAD)
    ycat, invin = _prep_call(degp, x)
    T = _scatter_kernel(ycat, ridx, col_r)

    ic = slice(0, C)
    wstack = jnp.stack([
        W_z[0, 0, ic], W_z[1, 0, ic], W_z[0, 1, ic], W_z[1, 1, ic],
        W_h[0, 0, ic], W_h[1, 0, ic], W_h[0, 1, ic], W_h[1, 1, ic],
    ])
    bstack = jnp.stack([b_z, b_h])
    return _mm_call(x, T, invin, wstack, bstack, lin_W,
                    lin_b.reshape(1, C))


# speedup vs baseline: 1.1892x; 1.0012x over previous
import functools

import jax
import jax.numpy as jnp
from jax import lax
from jax.experimental import pallas as pl
from jax.experimental.pallas import tpu as pltpu
from jax.experimental.pallas import tpu_sc as plsc

N = 10000
E = 320000
C = 128
NC = 2
NS = 16
NPAD = 10240
ROWS_PER_TILE = NPAD // NS
EDGES_PER_TILE = E // NS
CH = 80
CHN = EDGES_PER_TILE // CH
CHB = 10
NSTAGE = CHN // CHB

_MESH = dict(core_axis_name="c", subcore_axis_name="s", num_cores=NC,
             num_subcores=NS)


@functools.partial(
    pl.kernel,
    out_type=jax.ShapeDtypeStruct((NC * NS * NPAD,), jnp.float32),
    mesh=plsc.VectorSubcoreMesh(**_MESH),
    scratch_types=[
        pltpu.VMEM((EDGES_PER_TILE,), jnp.int32),
        pltpu.VMEM((EDGES_PER_TILE,), jnp.float32),
        pltpu.VMEM((NPAD,), jnp.float32),
    ],
    compiler_params=pltpu.CompilerParams(needs_layout_passes=False),
)
def _deg_kernel(eiflat_hbm, ew_hbm, degp_hbm, idx_v, w_v, acc_v):
    c = lax.axis_index("c")
    s = lax.axis_index("s")
    base = s * EDGES_PER_TILE

    def zero(i, _):
        acc_v[pl.ds(i * 16, 16)] = jnp.zeros((16,), jnp.float32)
        return 0
    lax.fori_loop(0, NPAD // 16, zero, 0)

    pltpu.sync_copy(eiflat_hbm.at[pl.ds(c * E + base, EDGES_PER_TILE)], idx_v)
    pltpu.sync_copy(ew_hbm.at[pl.ds(base, EDGES_PER_TILE)], w_v)

    def body(i, _):
        for u in range(2):
            idx = idx_v[pl.ds(i * 32 + u * 16, 16)]
            w = w_v[pl.ds(i * 32 + u * 16, 16)]
            plsc.addupdate_scatter(acc_v, [idx], w)
        return 0
    lax.fori_loop(0, EDGES_PER_TILE // 32, body, 0)

    pltpu.sync_copy(acc_v,
                    degp_hbm.at[pl.ds((c * NS + s) * NPAD, NPAD)])


def _prep_body(degp_ref, x_ref, ycat_ref, invin_ref):
    deg = jnp.sum(degp_ref[...], axis=1)
    dgo = deg[0, :N]
    dgi = deg[1, :N]
    invo = jnp.where(dgo != 0.0, 1.0 / dgo, 0.0)
    invi = jnp.where(dgi != 0.0, 1.0 / dgi, 0.0)
    xv = x_ref[...]
    ycat_ref[0:N, :] = xv * invo[:, None]
    ycat_ref[N:2 * N, :] = xv
    invin_ref[...] = jnp.broadcast_to(invi[:, None], (N, 8))


def _prep_call(degp, x):
    return pl.pallas_call(
        _prep_body,
        out_shape=[jax.ShapeDtypeStruct((2 * N, C), jnp.float32),
                   jax.ShapeDtypeStruct((N, 8), jnp.float32)],
    )(degp, x)


@functools.partial(
    pl.kernel,
    out_type=jax.ShapeDtypeStruct((NC, NPAD, C), jnp.float32),
    mesh=plsc.VectorSubcoreMesh(**_MESH),
    scratch_types=[
        pltpu.VMEM_SHARED((NPAD, C), jnp.float32),
        pltpu.VMEM((CHB, CH), jnp.int32),
        pltpu.VMEM((CHB, CH), jnp.int32),
        pltpu.VMEM((CH, C), jnp.float32),
        pltpu.VMEM((CH, C), jnp.float32),
        pltpu.VMEM((CH, C), jnp.float32),
        pltpu.VMEM((64, C), jnp.float32),
        pltpu.SemaphoreType.DMA,
        pltpu.SemaphoreType.DMA,
        pltpu.SemaphoreType.DMA,
        pltpu.SemaphoreType.DMA,
        pltpu.SemaphoreType.DMA,
        pltpu.SemaphoreType.DMA,
    ],
    compiler_params=pltpu.CompilerParams(use_tc_tiling_on_sc=False),
)
def _scatter_kernel(ycat_hbm, ridx_hbm, col_hbm, out_hbm,
                    acc, rowi, coli, rbufa, rbufb, rbufc, zb,
                    gsa, gsb, gsc, ssa, ssb, ssc):
    c = lax.axis_index("c")
    s = lax.axis_index("s")

    def z16(i, _):
        r = i // (C // 16)
        f = (i % (C // 16)) * 16
        zb[r, pl.ds(f, 16)] = jnp.zeros((16,), jnp.float32)
        return 0
    lax.fori_loop(0, 64 * (C // 16), z16, 0)

    def zc(k, _):
        pltpu.sync_copy(zb, acc.at[pl.ds(s * ROWS_PER_TILE + k * 64, 64)])
        return 0
    lax.fori_loop(0, ROWS_PER_TILE // 64, zc, 0)

    plsc.subcore_barrier()

    bufs = (rbufa, rbufb, rbufc)
    gsems = (gsa, gsb, gsc)
    ssems = (ssa, ssb, ssc)

    def stage(g, _):
        pltpu.sync_copy(ridx_hbm.at[c, s, pl.ds(g * CHB, CHB)], rowi)
        pltpu.sync_copy(col_hbm.at[s, pl.ds(g * CHB, CHB)], coli)

        gd, sd = {}, {}

        def scat(j):
            gd[j].wait()
            sd[j] = pltpu.async_copy(bufs[j % 3], acc.at[coli.at[j]],
                                     ssems[j % 3], add=True)

        for k in range(CHB):
            if k - 3 >= 0:
                sd[k - 3].wait()
            gd[k] = pltpu.async_copy(ycat_hbm.at[rowi.at[k]],
                                     bufs[k % 3], gsems[k % 3])
            if k - 2 >= 0:
                scat(k - 2)
        scat(CHB - 2)
        scat(CHB - 1)
        for j in range(CHB - 3, CHB):
            sd[j].wait()
        return 0
    lax.fori_loop(0, NSTAGE, stage, 0)
    plsc.subcore_barrier()

    pltpu.sync_copy(acc.at[pl.ds(s * ROWS_PER_TILE, ROWS_PER_TILE)],
                    out_hbm.at[c, pl.ds(s * ROWS_PER_TILE, ROWS_PER_TILE)])


_R = 2000


def _mm_body(x_ref, to_ref, s_ref, invin_ref, w_ref, b_ref, lw_ref, lb_ref,
             o_ref):
    xb = x_ref[...]
    tob = to_ref[0]
    tib = s_ref[0] * invin_ref[:, 0:1]
    w = w_ref[...]
    dot = functools.partial(jnp.dot, preferred_element_type=jnp.float32)
    az = (dot(xb, w[0] + w[1]) + dot(tob, w[2]) + dot(tib, w[3])
          + b_ref[0][None, :])
    ah = (dot(xb, w[4] + w[5]) + dot(tob, w[6]) + dot(tib, w[7])
          + b_ref[1][None, :])
    z = jax.nn.sigmoid(az)
    h = jnp.maximum((1.0 - z) * jnp.tanh(ah), 0.0)
    o_ref[...] = lax.dot_general(h, lw_ref[...], (((1,), (1,)), ((), ())),
                                 preferred_element_type=jnp.float32) \
        + lb_ref[0][None, :]


def _mm_call(x, T, invin, wstack, bstack, lin_W, lin_b):
    grid = N // _R
    return pl.pallas_call(
        _mm_body,
        grid=(grid,),
        in_specs=[
            pl.BlockSpec((_R, C), lambda i: (i, 0)),
            pl.BlockSpec((1, _R, C), lambda i: (0, i, 0)),
            pl.BlockSpec((1, _R, C), lambda i: (1, i, 0)),
            pl.BlockSpec((_R, 8), lambda i: (i, 0)),
            pl.BlockSpec((8, C, C), lambda i: (0, 0, 0)),
            pl.BlockSpec((2, C), lambda i: (0, 0)),
            pl.BlockSpec((C, C), lambda i: (0, 0)),
            pl.BlockSpec((1, C), lambda i: (0, 0)),
        ],
        out_specs=pl.BlockSpec((_R, C), lambda i: (i, 0)),
        out_shape=jax.ShapeDtypeStruct((N, C), jnp.float32),
    )(x, T, T, invin, wstack, bstack, lin_W, lin_b)


def kernel(x, edge_index, edge_weight, W_z, b_z, W_r, b_r, W_h, b_h,
           lin_W, lin_b):
    del W_r, b_r
    row_r = edge_index[0].reshape(NS, CHN, CH)
    col_r = edge_index[1].reshape(NS, CHN, CH)
    ridx = jnp.stack([row_r, row_r + N])

    degp = _deg_kernel(edge_index.reshape(-1),
                       edge_weight).reshape(NC, NS, NPAD)
    ycat, invin = _prep_call(degp, x)
    T = _scatter_kernel(ycat, ridx, col_r)

    ic = slice(0, C)
    wstack = jnp.stack([
        W_z[0, 0, ic], W_z[1, 0, ic], W_z[0, 1, ic], W_z[1, 1, ic],
        W_h[0, 0, ic], W_h[1, 0, ic], W_h[0, 1, ic], W_h[1, 1, ic],
    ])
    bstack = jnp.stack([b_z, b_h])
    return _mm_call(x, T, invin, wstack, bstack, lin_W,
                    lin_b.reshape(1, C))


# speedup vs baseline: 1.3616x; 1.1449x over previous
import functools

import jax
import jax.numpy as jnp
from jax import lax
from jax.experimental import pallas as pl
from jax.experimental.pallas import tpu as pltpu
from jax.experimental.pallas import tpu_sc as plsc

N = 10000
E = 320000
C = 128
NC = 2
NS = 16
NPAD = 10240
ROWS_PER_TILE = NPAD // NS
EDGES_PER_TILE = E // NS
CH = 80
CHN = EDGES_PER_TILE // CH
CHB = 25
NSTAGE = CHN // CHB

_MESH = dict(core_axis_name="c", subcore_axis_name="s", num_cores=NC,
             num_subcores=NS)


@functools.partial(
    pl.kernel,
    out_type=jax.ShapeDtypeStruct((NC * NS * NPAD,), jnp.float32),
    mesh=plsc.VectorSubcoreMesh(**_MESH),
    scratch_types=[
        pltpu.VMEM((EDGES_PER_TILE,), jnp.int32),
        pltpu.VMEM((EDGES_PER_TILE,), jnp.float32),
        pltpu.VMEM((NPAD,), jnp.float32),
    ],
    compiler_params=pltpu.CompilerParams(needs_layout_passes=False),
)
def _deg_kernel(eiflat_hbm, ew_hbm, degp_hbm, idx_v, w_v, acc_v):
    c = lax.axis_index("c")
    s = lax.axis_index("s")
    base = s * EDGES_PER_TILE

    def zero(i, _):
        acc_v[pl.ds(i * 16, 16)] = jnp.zeros((16,), jnp.float32)
        return 0
    lax.fori_loop(0, NPAD // 16, zero, 0)

    pltpu.sync_copy(eiflat_hbm.at[pl.ds(c * E + base, EDGES_PER_TILE)], idx_v)
    pltpu.sync_copy(ew_hbm.at[pl.ds(base, EDGES_PER_TILE)], w_v)

    def body(i, _):
        for u in range(2):
            idx = idx_v[pl.ds(i * 32 + u * 16, 16)]
            w = w_v[pl.ds(i * 32 + u * 16, 16)]
            plsc.addupdate_scatter(acc_v, [idx], w)
        return 0
    lax.fori_loop(0, EDGES_PER_TILE // 32, body, 0)

    pltpu.sync_copy(acc_v,
                    degp_hbm.at[pl.ds((c * NS + s) * NPAD, NPAD)])


def _prep_body(degp_ref, x_ref, ycat_ref, invin_ref):
    deg = jnp.sum(degp_ref[...], axis=1)
    dgo = deg[0, :N]
    dgi = deg[1, :N]
    invo = jnp.where(dgo != 0.0, 1.0 / dgo, 0.0)
    invi = jnp.where(dgi != 0.0, 1.0 / dgi, 0.0)
    xv = x_ref[...]
    ycat_ref[0:N, :] = xv * invo[:, None]
    ycat_ref[N:2 * N, :] = xv
    invin_ref[...] = jnp.broadcast_to(invi[:, None], (N, 8))


def _prep_call(degp, x):
    return pl.pallas_call(
        _prep_body,
        out_shape=[jax.ShapeDtypeStruct((2 * N, C), jnp.float32),
                   jax.ShapeDtypeStruct((N, 8), jnp.float32)],
    )(degp, x)


@functools.partial(
    pl.kernel,
    out_type=jax.ShapeDtypeStruct((NC, NPAD, C), jnp.float32),
    mesh=plsc.VectorSubcoreMesh(**_MESH),
    scratch_types=[
        pltpu.VMEM_SHARED((NPAD, C), jnp.float32),
        pltpu.VMEM((CHB, CH), jnp.int32),
        pltpu.VMEM((CHB, CH), jnp.int32),
        pltpu.VMEM((CH, C), jnp.float32),
        pltpu.VMEM((CH, C), jnp.float32),
        pltpu.VMEM((CH, C), jnp.float32),
        pltpu.VMEM((64, C), jnp.float32),
        pltpu.SemaphoreType.DMA,
        pltpu.SemaphoreType.DMA,
        pltpu.SemaphoreType.DMA,
        pltpu.SemaphoreType.DMA,
        pltpu.SemaphoreType.DMA,
        pltpu.SemaphoreType.DMA,
    ],
    compiler_params=pltpu.CompilerParams(use_tc_tiling_on_sc=False),
)
def _scatter_kernel(ycat_hbm, ridx_hbm, col_hbm, out_hbm,
                    acc, rowi, coli, rbufa, rbufb, rbufc, zb,
                    gsa, gsb, gsc, ssa, ssb, ssc):
    c = lax.axis_index("c")
    s = lax.axis_index("s")

    def z16(i, _):
        r = i // (C // 16)
        f = (i % (C // 16)) * 16
        zb[r, pl.ds(f, 16)] = jnp.zeros((16,), jnp.float32)
        return 0
    lax.fori_loop(0, 64 * (C // 16), z16, 0)

    def zc(k, _):
        pltpu.sync_copy(zb, acc.at[pl.ds(s * ROWS_PER_TILE + k * 64, 64)])
        return 0
    lax.fori_loop(0, ROWS_PER_TILE // 64, zc, 0)

    plsc.subcore_barrier()

    bufs = (rbufa, rbufb, rbufc)
    gsems = (gsa, gsb, gsc)
    ssems = (ssa, ssb, ssc)

    def stage(g, _):
        pltpu.sync_copy(ridx_hbm.at[c, s, pl.ds(g * CHB, CHB)], rowi)
        pltpu.sync_copy(col_hbm.at[s, pl.ds(g * CHB, CHB)], coli)

        gd, sd = {}, {}

        def scat(j):
            gd[j].wait()
            sd[j] = pltpu.async_copy(bufs[j % 3], acc.at[coli.at[j]],
                                     ssems[j % 3], add=True)

        for k in range(CHB):
            if k - 3 >= 0:
                sd[k - 3].wait()
            gd[k] = pltpu.async_copy(ycat_hbm.at[rowi.at[k]],
                                     bufs[k % 3], gsems[k % 3])
            if k - 2 >= 0:
                scat(k - 2)
        scat(CHB - 2)
        scat(CHB - 1)
        for j in range(CHB - 3, CHB):
            sd[j].wait()
        return 0
    lax.fori_loop(0, NSTAGE, stage, 0)
    plsc.subcore_barrier()

    pltpu.sync_copy(acc.at[pl.ds(s * ROWS_PER_TILE, ROWS_PER_TILE)],
                    out_hbm.at[c, pl.ds(s * ROWS_PER_TILE, ROWS_PER_TILE)])


_R = 2000


def _mm_body(x_ref, to_ref, s_ref, invin_ref, w_ref, b_ref, lw_ref, lb_ref,
             o_ref):
    xb = x_ref[...]
    tob = to_ref[0]
    tib = s_ref[0] * invin_ref[:, 0:1]
    w = w_ref[...]
    dot = functools.partial(jnp.dot, preferred_element_type=jnp.float32)
    az = (dot(xb, w[0] + w[1]) + dot(tob, w[2]) + dot(tib, w[3])
          + b_ref[0][None, :])
    ah = (dot(xb, w[4] + w[5]) + dot(tob, w[6]) + dot(tib, w[7])
          + b_ref[1][None, :])
    z = jax.nn.sigmoid(az)
    h = jnp.maximum((1.0 - z) * jnp.tanh(ah), 0.0)
    o_ref[...] = lax.dot_general(h, lw_ref[...], (((1,), (1,)), ((), ())),
                                 preferred_element_type=jnp.float32) \
        + lb_ref[0][None, :]


def _mm_call(x, T, invin, wstack, bstack, lin_W, lin_b):
    grid = N // _R
    return pl.pallas_call(
        _mm_body,
        grid=(grid,),
        in_specs=[
            pl.BlockSpec((_R, C), lambda i: (i, 0)),
            pl.BlockSpec((1, _R, C), lambda i: (0, i, 0)),
            pl.BlockSpec((1, _R, C), lambda i: (1, i, 0)),
            pl.BlockSpec((_R, 8), lambda i: (i, 0)),
            pl.BlockSpec((8, C, C), lambda i: (0, 0, 0)),
            pl.BlockSpec((2, C), lambda i: (0, 0)),
            pl.BlockSpec((C, C), lambda i: (0, 0)),
            pl.BlockSpec((1, C), lambda i: (0, 0)),
        ],
        out_specs=pl.BlockSpec((_R, C), lambda i: (i, 0)),
        out_shape=jax.ShapeDtypeStruct((N, C), jnp.float32),
    )(x, T, T, invin, wstack, bstack, lin_W, lin_b)


def kernel(x, edge_index, edge_weight, W_z, b_z, W_r, b_r, W_h, b_h,
           lin_W, lin_b):
    del W_r, b_r
    row_r = edge_index[0].reshape(NS, CHN, CH)
    col_r = edge_index[1].reshape(NS, CHN, CH)
    ridx = jnp.stack([row_r, row_r + N])

    degp = _deg_kernel(edge_index.reshape(-1),
                       edge_weight).reshape(NC, NS, NPAD)
    ycat, invin = _prep_call(degp, x)
    T = _scatter_kernel(ycat, ridx, col_r)

    ic = slice(0, C)
    wstack = jnp.stack([
        W_z[0, 0, ic], W_z[1, 0, ic], W_z[0, 1, ic], W_z[1, 1, ic],
        W_h[0, 0, ic], W_h[1, 0, ic], W_h[0, 1, ic], W_h[1, 1, ic],
    ])
    bstack = jnp.stack([b_z, b_h])
    return _mm_call(x, T, invin, wstack, bstack, lin_W,
                    lin_b.reshape(1, C))


# speedup vs baseline: 1.4204x; 1.0432x over previous
import functools

import jax
import jax.numpy as jnp
from jax import lax
from jax.experimental import pallas as pl
from jax.experimental.pallas import tpu as pltpu
from jax.experimental.pallas import tpu_sc as plsc

N = 10000
E = 320000
C = 128
NC = 2
NS = 16
NPAD = 10240
ROWS_PER_TILE = NPAD // NS
EDGES_PER_TILE = E // NS
CH = 80
CHN = EDGES_PER_TILE // CH
CHB = 50
NSTAGE = CHN // CHB

_MESH = dict(core_axis_name="c", subcore_axis_name="s", num_cores=NC,
             num_subcores=NS)


@functools.partial(
    pl.kernel,
    out_type=jax.ShapeDtypeStruct((NC * NS * NPAD,), jnp.float32),
    mesh=plsc.VectorSubcoreMesh(**_MESH),
    scratch_types=[
        pltpu.VMEM((EDGES_PER_TILE,), jnp.int32),
        pltpu.VMEM((EDGES_PER_TILE,), jnp.float32),
        pltpu.VMEM((NPAD,), jnp.float32),
    ],
    compiler_params=pltpu.CompilerParams(needs_layout_passes=False),
)
def _deg_kernel(eiflat_hbm, ew_hbm, degp_hbm, idx_v, w_v, acc_v):
    c = lax.axis_index("c")
    s = lax.axis_index("s")
    base = s * EDGES_PER_TILE

    def zero(i, _):
        acc_v[pl.ds(i * 16, 16)] = jnp.zeros((16,), jnp.float32)
        return 0
    lax.fori_loop(0, NPAD // 16, zero, 0)

    pltpu.sync_copy(eiflat_hbm.at[pl.ds(c * E + base, EDGES_PER_TILE)], idx_v)
    pltpu.sync_copy(ew_hbm.at[pl.ds(base, EDGES_PER_TILE)], w_v)

    def body(i, _):
        for u in range(2):
            idx = idx_v[pl.ds(i * 32 + u * 16, 16)]
            w = w_v[pl.ds(i * 32 + u * 16, 16)]
            plsc.addupdate_scatter(acc_v, [idx], w)
        return 0
    lax.fori_loop(0, EDGES_PER_TILE // 32, body, 0)

    pltpu.sync_copy(acc_v,
                    degp_hbm.at[pl.ds((c * NS + s) * NPAD, NPAD)])


def _prep_body(degp_ref, x_ref, ycat_ref, invin_ref):
    deg = jnp.sum(degp_ref[...], axis=1)
    dgo = deg[0, :N]
    dgi = deg[1, :N]
    invo = jnp.where(dgo != 0.0, 1.0 / dgo, 0.0)
    invi = jnp.where(dgi != 0.0, 1.0 / dgi, 0.0)
    xv = x_ref[...]
    ycat_ref[0:N, :] = xv * invo[:, None]
    ycat_ref[N:2 * N, :] = xv
    invin_ref[...] = jnp.broadcast_to(invi[:, None], (N, 8))


def _prep_call(degp, x):
    return pl.pallas_call(
        _prep_body,
        out_shape=[jax.ShapeDtypeStruct((2 * N, C), jnp.float32),
                   jax.ShapeDtypeStruct((N, 8), jnp.float32)],
    )(degp, x)


@functools.partial(
    pl.kernel,
    out_type=jax.ShapeDtypeStruct((NC, NPAD, C), jnp.float32),
    mesh=plsc.VectorSubcoreMesh(**_MESH),
    scratch_types=[
        pltpu.VMEM_SHARED((NPAD, C), jnp.float32),
        pltpu.VMEM((CHB, CH), jnp.int32),
        pltpu.VMEM((CHB, CH), jnp.int32),
        pltpu.VMEM((CH, C), jnp.float32),
        pltpu.VMEM((CH, C), jnp.float32),
        pltpu.VMEM((CH, C), jnp.float32),
        pltpu.VMEM((64, C), jnp.float32),
        pltpu.SemaphoreType.DMA,
        pltpu.SemaphoreType.DMA,
        pltpu.SemaphoreType.DMA,
        pltpu.SemaphoreType.DMA,
        pltpu.SemaphoreType.DMA,
        pltpu.SemaphoreType.DMA,
    ],
    compiler_params=pltpu.CompilerParams(use_tc_tiling_on_sc=False),
)
def _scatter_kernel(ycat_hbm, ridx_hbm, col_hbm, out_hbm,
                    acc, rowi, coli, rbufa, rbufb, rbufc, zb,
                    gsa, gsb, gsc, ssa, ssb, ssc):
    c = lax.axis_index("c")
    s = lax.axis_index("s")

    def z16(i, _):
        r = i // (C // 16)
        f = (i % (C // 16)) * 16
        zb[r, pl.ds(f, 16)] = jnp.zeros((16,), jnp.float32)
        return 0
    lax.fori_loop(0, 64 * (C // 16), z16, 0)

    def zc(k, _):
        pltpu.sync_copy(zb, acc.at[pl.ds(s * ROWS_PER_TILE + k * 64, 64)])
        return 0
    lax.fori_loop(0, ROWS_PER_TILE // 64, zc, 0)

    plsc.subcore_barrier()

    bufs = (rbufa, rbufb, rbufc)
    gsems = (gsa, gsb, gsc)
    ssems = (ssa, ssb, ssc)

    def stage(g, _):
        pltpu.sync_copy(ridx_hbm.at[c, s, pl.ds(g * CHB, CHB)], rowi)
        pltpu.sync_copy(col_hbm.at[s, pl.ds(g * CHB, CHB)], coli)

        gd, sd = {}, {}

        def scat(j):
            gd[j].wait()
            sd[j] = pltpu.async_copy(bufs[j % 3], acc.at[coli.at[j]],
                                     ssems[j % 3], add=True)

        for k in range(CHB):
            if k - 3 >= 0:
                sd[k - 3].wait()
            gd[k] = pltpu.async_copy(ycat_hbm.at[rowi.at[k]],
                                     bufs[k % 3], gsems[k % 3])
            if k - 2 >= 0:
                scat(k - 2)
        scat(CHB - 2)
        scat(CHB - 1)
        for j in range(CHB - 3, CHB):
            sd[j].wait()
        return 0
    lax.fori_loop(0, NSTAGE, stage, 0)
    plsc.subcore_barrier()

    pltpu.sync_copy(acc.at[pl.ds(s * ROWS_PER_TILE, ROWS_PER_TILE)],
                    out_hbm.at[c, pl.ds(s * ROWS_PER_TILE, ROWS_PER_TILE)])


_R = 2000


def _mm_body(x_ref, to_ref, s_ref, invin_ref, w_ref, b_ref, lw_ref, lb_ref,
             o_ref):
    xb = x_ref[...]
    tob = to_ref[0]
    tib = s_ref[0] * invin_ref[:, 0:1]
    w = w_ref[...]
    dot = functools.partial(jnp.dot, preferred_element_type=jnp.float32)
    az = (dot(xb, w[0] + w[1]) + dot(tob, w[2]) + dot(tib, w[3])
          + b_ref[0][None, :])
    ah = (dot(xb, w[4] + w[5]) + dot(tob, w[6]) + dot(tib, w[7])
          + b_ref[1][None, :])
    z = jax.nn.sigmoid(az)
    h = jnp.maximum((1.0 - z) * jnp.tanh(ah), 0.0)
    o_ref[...] = lax.dot_general(h, lw_ref[...], (((1,), (1,)), ((), ())),
                                 preferred_element_type=jnp.float32) \
        + lb_ref[0][None, :]


def _mm_call(x, T, invin, wstack, bstack, lin_W, lin_b):
    grid = N // _R
    return pl.pallas_call(
        _mm_body,
        grid=(grid,),
        in_specs=[
            pl.BlockSpec((_R, C), lambda i: (i, 0)),
            pl.BlockSpec((1, _R, C), lambda i: (0, i, 0)),
            pl.BlockSpec((1, _R, C), lambda i: (1, i, 0)),
            pl.BlockSpec((_R, 8), lambda i: (i, 0)),
            pl.BlockSpec((8, C, C), lambda i: (0, 0, 0)),
            pl.BlockSpec((2, C), lambda i: (0, 0)),
            pl.BlockSpec((C, C), lambda i: (0, 0)),
            pl.BlockSpec((1, C), lambda i: (0, 0)),
        ],
        out_specs=pl.BlockSpec((_R, C), lambda i: (i, 0)),
        out_shape=jax.ShapeDtypeStruct((N, C), jnp.float32),
    )(x, T, T, invin, wstack, bstack, lin_W, lin_b)


def kernel(x, edge_index, edge_weight, W_z, b_z, W_r, b_r, W_h, b_h,
           lin_W, lin_b):
    del W_r, b_r
    row_r = edge_index[0].reshape(NS, CHN, CH)
    col_r = edge_index[1].reshape(NS, CHN, CH)
    ridx = jnp.stack([row_r, row_r + N])

    degp = _deg_kernel(edge_index.reshape(-1),
                       edge_weight).reshape(NC, NS, NPAD)
    ycat, invin = _prep_call(degp, x)
    T = _scatter_kernel(ycat, ridx, col_r)

    ic = slice(0, C)
    wstack = jnp.stack([
        W_z[0, 0, ic], W_z[1, 0, ic], W_z[0, 1, ic], W_z[1, 1, ic],
        W_h[0, 0, ic], W_h[1, 0, ic], W_h[0, 1, ic], W_h[1, 1, ic],
    ])
    bstack = jnp.stack([b_z, b_h])
    return _mm_call(x, T, invin, wstack, bstack, lin_W,
                    lin_b.reshape(1, C))
